# baseline (speedup 1.0000x reference)
import functools

import jax
import jax.numpy as jnp
from jax import lax
from jax.experimental import pallas as pl
from jax.experimental.pallas import tpu as pltpu
from jax.experimental.pallas import tpu_sc as plsc

_E = 8
_NEG_INF = float("-inf")
_T = 2048


def _logits_body(x_ref, gw_ref, out_ref):
    out_ref[...] = lax.dot_general(gw_ref[...], x_ref[...],
                                   (((1,), (1,)), ((), ())),
                                   preferred_element_type=jnp.float32)


def _route_body(lg_hbm, dw_hbm, lg_ref, dw_ref):
    info = plsc.get_sparse_core_info()
    nc = info.num_cores
    wid = lax.axis_index("s") * nc + lax.axis_index("c")
    base = wid * 64
    for e in range(_E):
        pltpu.sync_copy(lg_hbm.at[pl.ds(e * _T + base, 64)],
                        lg_ref.at[pl.ds(e * 64, 64)])
    for g in range(4):
        l0 = lg_ref[pl.ds(g * 16, 16)]
        m1 = l0
        i1 = jnp.zeros((16,), jnp.int32)
        m2 = jnp.full((16,), _NEG_INF, jnp.float32)
        i2 = jnp.zeros((16,), jnp.int32)
        for e in range(1, _E):
            le = lg_ref[pl.ds(e * 64 + g * 16, 16)]
            gt1 = le > m1
            cand_m2 = jnp.where(le > m2, le, m2)
            cand_i2 = jnp.where(le > m2, e, i2)
            m2 = jnp.where(gt1, m1, cand_m2)
            i2 = jnp.where(gt1, i1, cand_i2)
            m1 = jnp.where(gt1, le, m1)
            i1 = jnp.where(gt1, e, i1)
        s = jnp.exp(m2 - m1)
        w1 = 1.0 / (1.0 + s)
        w2 = 1.0 - w1
        zero = jnp.zeros((16,), jnp.float32)
        for e in range(_E):
            col = (jnp.where(i1 == e, w1, zero)
                   + jnp.where(i2 == e, w2, zero))
            dw_ref[pl.ds(e * 64 + g * 16, 16)] = col
    for e in range(_E):
        pltpu.sync_copy(dw_ref.at[pl.ds(e * 64, 64)],
                        dw_hbm.at[pl.ds(e * _T + base, 64)])


def _moe_body(x_ref, wcol_ref, b_ref, a_ref, out_ref, xb_ref):
    e = pl.program_id(0)

    @pl.when(e == 0)
    def _prep():
        xb_ref[...] = x_ref[...].astype(jnp.bfloat16)

    abf = a_ref[0].astype(jnp.bfloat16)
    y = jnp.dot(xb_ref[...], abf, preferred_element_type=jnp.float32)
    contrib = wcol_ref[0] * (y + b_ref[0])

    @pl.when(e == 0)
    def _init():
        out_ref[...] = contrib

    @pl.when(e != 0)
    def _acc():
        out_ref[...] += contrib


@functools.partial(jax.jit, static_argnames=())
def kernel(inputs, gate_w, expert_A, expert_b):
    batch_shape = inputs.shape[:-1]
    d = inputs.shape[-1]
    x = inputs.reshape(-1, d)
    t = x.shape[0]

    logits_t = pl.pallas_call(
        _logits_body,
        in_specs=[pl.BlockSpec((t, d), lambda: (0, 0)),
                  pl.BlockSpec((_E, d), lambda: (0, 0))],
        out_specs=pl.BlockSpec((_E, t), lambda: (0, 0)),
        out_shape=jax.ShapeDtypeStruct((_E, t), jnp.float32),
    )(x, gate_w).reshape(_E * t)

    route = pl.kernel(
        _route_body,
        out_type=jax.ShapeDtypeStruct((t * _E,), jnp.float32),
        mesh=plsc.VectorSubcoreMesh(core_axis_name="c", subcore_axis_name="s"),
        scratch_types=[pltpu.VMEM((64 * _E,), jnp.float32),
                       pltpu.VMEM((64 * _E,), jnp.float32)],
    )
    wcols = route(logits_t).reshape(_E, t, 1)

    out = pl.pallas_call(
        _moe_body,
        grid=(_E,),
        in_specs=[
            pl.BlockSpec((t, d), lambda e: (0, 0)),
            pl.BlockSpec((1, t, 1), lambda e: (e, 0, 0)),
            pl.BlockSpec((1, 1, d), lambda e: (e, 0, 0)),
            pl.BlockSpec((1, d, d), lambda e: (e, 0, 0)),
        ],
        out_specs=pl.BlockSpec((t, d), lambda e: (0, 0)),
        out_shape=jax.ShapeDtypeStruct((t, d), jnp.float32),
        scratch_shapes=[
            pltpu.VMEM((t, d), jnp.bfloat16),
        ],
    )(x, wcols, expert_b.reshape(_E, 1, d), expert_A)
    return out.reshape(*batch_shape, d)


# speedup vs baseline: 1.9714x; 1.9714x over previous
import functools

import jax
import jax.numpy as jnp
from jax import lax
from jax.experimental import pallas as pl
from jax.experimental.pallas import tpu as pltpu

_E = 8
_NEG_INF = float("-inf")


def _moe_body(x_ref, gw_ref, b_ref, a_ref, out_ref, xb_ref, wcol_ref):
    e = pl.program_id(0)

    @pl.when(e == 0)
    def _routing():
        xt = x_ref[...]
        xb_ref[...] = xt.astype(jnp.bfloat16)
        logits = lax.dot_general(xt, gw_ref[...],
                                 (((1,), (1,)), ((), ())),
                                 preferred_element_type=jnp.float32)
        lane = jax.lax.broadcasted_iota(jnp.int32, logits.shape, 1)
        m1 = jnp.max(logits, axis=1, keepdims=True)
        idx0 = jnp.min(jnp.where(logits == m1, lane, _E), axis=1,
                       keepdims=True)
        logits2 = jnp.where(lane == idx0, _NEG_INF, logits)
        m2 = jnp.max(logits2, axis=1, keepdims=True)
        idx1 = jnp.min(jnp.where(logits2 == m2, lane, _E), axis=1,
                       keepdims=True)
        s = jnp.exp(m2 - m1)
        w0 = 1.0 / (1.0 + s)
        w1 = 1.0 - w0
        dense_w = (jnp.where(lane == idx0, w0, 0.0)
                   + jnp.where(lane == idx1, w1, 0.0))
        for ee in range(_E):
            wcol_ref[ee] = dense_w[:, ee:ee + 1]
        out_ref[...] = jnp.dot(dense_w, b_ref[...],
                               preferred_element_type=jnp.float32)

    abf = a_ref[0].astype(jnp.bfloat16)
    y = jnp.dot(xb_ref[...], abf, preferred_element_type=jnp.float32)
    out_ref[...] += wcol_ref[e] * y


@functools.partial(jax.jit, static_argnames=())
def kernel(inputs, gate_w, expert_A, expert_b):
    batch_shape = inputs.shape[:-1]
    d = inputs.shape[-1]
    x = inputs.reshape(-1, d)
    t = x.shape[0]

    out = pl.pallas_call(
        _moe_body,
        grid=(_E,),
        in_specs=[
            pl.BlockSpec((t, d), lambda e: (0, 0)),
            pl.BlockSpec((_E, d), lambda e: (0, 0)),
            pl.BlockSpec((_E, d), lambda e: (0, 0)),
            pl.BlockSpec((1, d, d), lambda e: (e, 0, 0)),
        ],
        out_specs=pl.BlockSpec((t, d), lambda e: (0, 0)),
        out_shape=jax.ShapeDtypeStruct((t, d), jnp.float32),
        scratch_shapes=[
            pltpu.VMEM((t, d), jnp.bfloat16),
            pltpu.VMEM((_E, t, 1), jnp.float32),
        ],
    )(x, gate_w, expert_b, expert_A)
    return out.reshape(*batch_shape, d)
